# baseline (speedup 1.0000x reference)
import math

import jax
import jax.numpy as jnp
from jax.experimental import pallas as pl

_Z_DIM = 10
_N = 1024


def _affine_consts():
    p = 0.75
    zs = []
    for k in range(_Z_DIM):
        geo = k * math.log(1.0 - p) + math.log(p)
        log_comb = (
            math.lgamma(_Z_DIM + 1.0)
            - math.lgamma(k + 1.0)
            - math.lgamma(_Z_DIM - k + 1.0)
        )
        zs.append(log_comb + geo)
    mx = max(zs)
    z = mx + math.log(sum(math.exp(v - mx) for v in zs))
    a = math.log(1.0 - p)
    b = math.log(p) - z
    return a, b


_A, _B = _affine_consts()


def _tc_kernel(zt_ref, zmt_ref, out_ref):
    sa = jnp.sum(zt_ref[...], axis=1, keepdims=True)
    sb = zmt_ref[0:1, :]
    for d in range(1, _Z_DIM):
        sb = sb + zmt_ref[d : d + 1, :]
    macc = jnp.minimum(zt_ref[:, 0:1], zmt_ref[0:1, :])
    for d in range(1, _Z_DIM):
        macc = macc + jnp.minimum(zt_ref[:, d : d + 1], zmt_ref[d : d + 1, :])
    dist = (sa + sb) - macc - macc
    k = jnp.floor(dist)
    out_ref[...] = k * _A + _B


def kernel(zt, ztm1, bi=1024):
    m = zt.shape[0]
    zmt = ztm1.T
    return pl.pallas_call(
        _tc_kernel,
        grid=(m // bi,),
        in_specs=[
            pl.BlockSpec((bi, _Z_DIM), lambda i: (i, 0)),
            pl.BlockSpec((_Z_DIM, _N), lambda i: (0, 0)),
        ],
        out_specs=pl.BlockSpec((bi, _N), lambda i: (i, 0)),
        out_shape=jax.ShapeDtypeStruct((m, _N), jnp.float32),
    )(zt, zmt)


# speedup vs baseline: 1.0128x; 1.0128x over previous
import math

import jax
import jax.numpy as jnp
from jax.experimental import pallas as pl

_Z_DIM = 10
_N = 1024


def _affine_consts():
    p = 0.75
    zs = []
    for k in range(_Z_DIM):
        geo = k * math.log(1.0 - p) + math.log(p)
        log_comb = (
            math.lgamma(_Z_DIM + 1.0)
            - math.lgamma(k + 1.0)
            - math.lgamma(_Z_DIM - k + 1.0)
        )
        zs.append(log_comb + geo)
    mx = max(zs)
    z = mx + math.log(sum(math.exp(v - mx) for v in zs))
    a = math.log(1.0 - p)
    b = math.log(p) - z
    return a, b


_A, _B = _affine_consts()


def _tc_kernel(zt_ref, zmt_ref, out_ref):
    sa = jnp.sum(zt_ref[...], axis=1, keepdims=True)
    sb = zmt_ref[0:1, :]
    for d in range(1, _Z_DIM):
        sb = sb + zmt_ref[d : d + 1, :]
    macc = jnp.minimum(zt_ref[:, 0:1], zmt_ref[0:1, :])
    for d in range(1, _Z_DIM):
        macc = macc + jnp.minimum(zt_ref[:, d : d + 1], zmt_ref[d : d + 1, :])
    dist = (sa + sb) - macc - macc
    k = jnp.floor(dist)
    out_ref[...] = k * _A + _B


def kernel(zt, ztm1, bi=512):
    m = zt.shape[0]
    zmt = ztm1.T
    return pl.pallas_call(
        _tc_kernel,
        grid=(m // bi,),
        in_specs=[
            pl.BlockSpec((bi, _Z_DIM), lambda i: (i, 0)),
            pl.BlockSpec((_Z_DIM, _N), lambda i: (0, 0)),
        ],
        out_specs=pl.BlockSpec((bi, _N), lambda i: (i, 0)),
        out_shape=jax.ShapeDtypeStruct((m, _N), jnp.float32),
    )(zt, zmt)
